# baseline (speedup 1.0000x reference)
import jax
import jax.numpy as jnp
from jax import lax
from jax.experimental import pallas as pl
from jax.experimental.pallas import tpu as pltpu

_VOCAB = 1000
_B_BLK = 64
_NBUF = 4


def _onehot_body(x_ref, o_ref, buf_ref, sem_ref):
    i = pl.program_id(0)
    nblk = pl.num_programs(0)
    b = lax.rem(i, _NBUF)
    h = x_ref.shape[1]

    @pl.when(i >= _NBUF)
    def _():
        pltpu.make_async_copy(
            buf_ref.at[b], o_ref.at[pl.ds((i - _NBUF) * _B_BLK, _B_BLK)],
            sem_ref.at[b],
        ).wait()

    idx = x_ref[...]
    col = lax.broadcasted_iota(jnp.int32, (_B_BLK, h, _VOCAB), 2)
    buf_ref[b] = (col == idx[:, :, None]).astype(jnp.float32)

    pltpu.make_async_copy(
        buf_ref.at[b], o_ref.at[pl.ds(i * _B_BLK, _B_BLK)], sem_ref.at[b]
    ).start()

    @pl.when(i == nblk - 1)
    def _():
        for k in range(_NBUF):
            j = i - (_NBUF - 1) + k
            bb = lax.rem(j, _NBUF)
            pltpu.make_async_copy(
                buf_ref.at[bb], o_ref.at[pl.ds(j * _B_BLK, _B_BLK)],
                sem_ref.at[bb],
            ).wait()


def kernel(x, W):
    b, h = x.shape
    nblk = b // _B_BLK
    return pl.pallas_call(
        _onehot_body,
        grid=(nblk,),
        in_specs=[pl.BlockSpec((_B_BLK, h), lambda i: (i, 0))],
        out_specs=pl.BlockSpec(memory_space=pl.ANY),
        out_shape=jax.ShapeDtypeStruct((b, h, _VOCAB), jnp.float32),
        scratch_shapes=[
            pltpu.VMEM((_NBUF, _B_BLK, h, _VOCAB), jnp.float32),
            pltpu.SemaphoreType.DMA((_NBUF,)),
        ],
    )(x)
